# Initial kernel scaffold; baseline (speedup 1.0000x reference)
import jax
import jax.numpy as jnp
from jax.experimental import pallas as pl


def kernel(x, Wr, br, W_in, gain, W_out, b_out):
    raise NotImplementedError("write your pallas kernel here")


# speedup vs baseline: 2.3131x; 2.3131x over previous
import functools

import jax
import jax.numpy as jnp
from jax.experimental import pallas as pl
from jax.experimental.pallas import tpu as pltpu

E = 8
TOP_K = 2
TILE_M = 256


def _router_body(x_ref, wr_ref, br_ref, idx_ref, gates_ref, zsq_ref):
    t = pl.program_id(0)
    logits = jnp.dot(x_ref[...], wr_ref[...],
                     preferred_element_type=jnp.float32) + br_ref[...]
    rows = logits.shape[0]
    lane = jax.lax.broadcasted_iota(jnp.int32, (rows, E), 1)
    v1 = jnp.max(logits, axis=-1, keepdims=True)
    i1 = jnp.min(jnp.where(logits == v1, lane, E), axis=-1, keepdims=True)
    masked = jnp.where(lane == i1, -jnp.inf, logits)
    v2 = jnp.max(masked, axis=-1, keepdims=True)
    i2 = jnp.min(jnp.where(masked == v2, lane, E), axis=-1, keepdims=True)
    e1 = jnp.exp(v2 - v1)
    denom = 1.0 + e1
    idx_ref[...] = jnp.concatenate([i1, i2], axis=-1)
    gates_ref[...] = jnp.concatenate([1.0 / denom, e1 / denom], axis=-1)
    z = v1 + jnp.log1p(e1)

    @pl.when(t == 0)
    def _():
        zsq_ref[...] = jnp.zeros_like(zsq_ref)

    zsq_ref[...] += jnp.full((1, 1), 1.0) * jnp.sum(z * z)


def _router(x2d, Wr, br):
    n = x2d.shape[0]
    d = x2d.shape[1]
    tile = 512
    grid = n // tile
    return pl.pallas_call(
        _router_body,
        grid=(grid,),
        in_specs=[
            pl.BlockSpec((tile, d), lambda t: (t, 0)),
            pl.BlockSpec((d, E), lambda t: (0, 0)),
            pl.BlockSpec((1, E), lambda t: (0, 0)),
        ],
        out_specs=[
            pl.BlockSpec((tile, TOP_K), lambda t: (t, 0)),
            pl.BlockSpec((tile, TOP_K), lambda t: (t, 0)),
            pl.BlockSpec((1, 1), lambda t: (0, 0)),
        ],
        out_shape=[
            jax.ShapeDtypeStruct((n, TOP_K), jnp.int32),
            jax.ShapeDtypeStruct((n, TOP_K), jnp.float32),
            jax.ShapeDtypeStruct((1, 1), jnp.float32),
        ],
    )(x2d, Wr, br.reshape(1, E))


def _expert_body(eot_ref, xg_ref, win_ref, gain_ref, wout_ref, bout_ref,
                 yg_ref):
    xt = xg_ref[...]
    h = jnp.dot(xt, win_ref[0], preferred_element_type=jnp.float32)
    d = xt.shape[1]
    x1 = h[:, :d]
    x2 = h[:, d:]
    x1 = 0.5 * x1 * (1.0 + jax.lax.erf(x1 * (2.0 ** -0.5)))
    xm = x1 * x2 * gain_ref[0]
    yg_ref[...] = (jnp.dot(xm, wout_ref[0], preferred_element_type=jnp.float32)
                   + bout_ref[0])


def _grouped_matmul(xg, eot, W_in, gain, W_out, b_out):
    pad_n, d = xg.shape
    nt = pad_n // TILE_M
    grid_spec = pltpu.PrefetchScalarGridSpec(
        num_scalar_prefetch=1,
        grid=(nt,),
        in_specs=[
            pl.BlockSpec((TILE_M, d), lambda t, eot: (t, 0)),
            pl.BlockSpec((1, d, 2 * d), lambda t, eot: (eot[t], 0, 0)),
            pl.BlockSpec((1, 1, d), lambda t, eot: (eot[t], 0, 0)),
            pl.BlockSpec((1, d, d), lambda t, eot: (eot[t], 0, 0)),
            pl.BlockSpec((1, 1, d), lambda t, eot: (eot[t], 0, 0)),
        ],
        out_specs=pl.BlockSpec((TILE_M, d), lambda t, eot: (t, 0)),
    )
    return pl.pallas_call(
        _expert_body,
        grid_spec=grid_spec,
        out_shape=jax.ShapeDtypeStruct((pad_n, d), jnp.float32),
        compiler_params=pltpu.CompilerParams(
            dimension_semantics=("arbitrary",)),
    )(eot, xg, W_in, gain.reshape(E, 1, d), W_out, b_out.reshape(E, 1, d))


def _combine_body(y0_ref, y1_ref, gates_ref, out_ref):
    g = gates_ref[...]
    out_ref[...] = g[:, 0:1] * y0_ref[...] + g[:, 1:2] * y1_ref[...]


def _combine(ygar, gates, n, d):
    tile = 512
    grid = n // tile
    nblk = n // tile
    return pl.pallas_call(
        _combine_body,
        grid=(grid,),
        in_specs=[
            pl.BlockSpec((tile, d), lambda t: (t, 0)),
            pl.BlockSpec((tile, d), lambda t, nblk=nblk: (nblk + t, 0)),
            pl.BlockSpec((tile, TOP_K), lambda t: (t, 0)),
        ],
        out_specs=pl.BlockSpec((tile, d), lambda t: (t, 0)),
        out_shape=jax.ShapeDtypeStruct((n, d), jnp.float32),
        compiler_params=pltpu.CompilerParams(
            dimension_semantics=("arbitrary",)),
    )(ygar, ygar, gates)


def kernel(x, Wr, br, W_in, gain, W_out, b_out):
    b, t, d = x.shape
    n = b * t
    na = n * TOP_K
    pad_n = na + E * TILE_M
    x2d = x.reshape(n, d)

    top_idx, gates, zsq = _router(x2d, Wr, br)
    z_loss = zsq[0, 0] / n

    ids = top_idx.reshape(-1)
    onehot = (ids[:, None] == jnp.arange(E, dtype=jnp.int32)[None, :])
    ranks = jnp.cumsum(onehot.astype(jnp.int32), axis=0) - 1
    rank = jnp.take_along_axis(ranks, ids[:, None], axis=1)[:, 0]
    counts = jnp.sum(onehot.astype(jnp.int32), axis=0)
    aligned = ((counts + TILE_M - 1) // TILE_M) * TILE_M
    off_full = jnp.concatenate(
        [jnp.zeros((1,), jnp.int32), jnp.cumsum(aligned)])
    slot = off_full[ids] + rank
    total = off_full[E]

    nt = pad_n // TILE_M
    tile_base = jnp.arange(nt, dtype=jnp.int32) * TILE_M
    eot = jnp.sum(tile_base[:, None] >= off_full[None, 1:], axis=1)
    eot = jnp.minimum(eot, E - 1).astype(jnp.int32)
    last_used = eot[jnp.maximum(total - 1, 0) // TILE_M]
    eot = jnp.where(tile_base < total, eot, last_used)

    tokens = (jnp.arange(na, dtype=jnp.int32) // TOP_K)
    sorted_token = jnp.zeros((pad_n,), jnp.int32).at[slot].set(tokens)
    xg = x2d[sorted_token]

    yg = _grouped_matmul(xg, eot, W_in, gain, W_out, b_out)

    islot_kn = slot.reshape(n, TOP_K).T.reshape(-1)
    ygar = yg[islot_kn]
    final = _combine(ygar, gates, n, d)

    return final.reshape(b, t, d), z_loss
